# Initial kernel scaffold; baseline (speedup 1.0000x reference)
import jax
import jax.numpy as jnp
from jax.experimental import pallas as pl


def kernel(a_x, m_x, a2a_edge_index, a2m_edge_index, m2a_edge_index, a2a_edge_weights, a2m_edge_weights, m2a_edge_weights, a2a_edge_attr, a2m_edge_attr, m2a_edge_attr, params):
    raise NotImplementedError("write your pallas kernel here")


# speedup vs baseline: 1.5301x; 1.5301x over previous
import functools
import math

import jax
import jax.numpy as jnp
from jax import lax
from jax.experimental import pallas as pl
from jax.experimental.pallas import tpu as pltpu
from jax.experimental.pallas import tpu_sc as plsc

D = 128
R = 50
NHEAD = 8
DH = D // NHEAD
LOG2 = math.log(2.0)
EPS = 1e-5

NW = 32
CHUNK = 128


def _ssp(x):
    return jnp.maximum(x, 0.0) + jnp.log(1.0 + jnp.exp(-jnp.abs(x))) - LOG2


def _ln(x, g, b):
    mu = jnp.mean(x, axis=-1, keepdims=True)
    var = jnp.mean((x - mu) ** 2, axis=-1, keepdims=True)
    return (x - mu) * lax.rsqrt(var + EPS) * g + b


def _pre_a_body(x_ref, g_ref, b_ref, w_ref, o_ref):
    x = _ln(x_ref[...], g_ref[...], b_ref[...])
    o_ref[...] = jnp.dot(x, w_ref[...], preferred_element_type=jnp.float32)


def _pre_a(x, g, b, w, block=1000):
    n = x.shape[0]
    grid = n // block
    return pl.pallas_call(
        _pre_a_body,
        grid=(grid,),
        in_specs=[
            pl.BlockSpec((block, D), lambda i: (i, 0)),
            pl.BlockSpec((1, D), lambda i: (0, 0)),
            pl.BlockSpec((1, D), lambda i: (0, 0)),
            pl.BlockSpec((D, D), lambda i: (0, 0)),
        ],
        out_specs=pl.BlockSpec((block, D), lambda i: (i, 0)),
        out_shape=jax.ShapeDtypeStruct((n, D), jnp.float32),
    )(x, g.reshape(1, D), b.reshape(1, D), w)


def _edge_w_body(n_valid, block, attr_ref, ew_ref, w1_ref, b1_ref, w2_ref,
                 b2_ref, o_ref):
    i = pl.program_id(0)
    t = _ssp(jnp.dot(attr_ref[...], w1_ref[...],
                     preferred_element_type=jnp.float32) + b1_ref[...])
    w = jnp.dot(t, w2_ref[...], preferred_element_type=jnp.float32) + b2_ref[...]
    c = 0.5 * (jnp.cos(ew_ref[...] * jnp.pi) + 1.0)
    w = w * c
    row = lax.broadcasted_iota(jnp.int32, (block, 1), 0) + i * block
    o_ref[...] = jnp.where(row < n_valid, w, 0.0)


def _edge_w(attr, ew, p, e_pad, n_valid, block=2048):
    grid = e_pad // block
    return pl.pallas_call(
        functools.partial(_edge_w_body, n_valid, block),
        grid=(grid,),
        in_specs=[
            pl.BlockSpec((block, R), lambda i: (i, 0)),
            pl.BlockSpec((block, 1), lambda i: (i, 0)),
            pl.BlockSpec((R, D), lambda i: (0, 0)),
            pl.BlockSpec((1, D), lambda i: (0, 0)),
            pl.BlockSpec((D, D), lambda i: (0, 0)),
            pl.BlockSpec((1, D), lambda i: (0, 0)),
        ],
        out_specs=pl.BlockSpec((block, D), lambda i: (i, 0)),
        out_shape=jax.ShapeDtypeStruct((e_pad, D), jnp.float32),
    )(attr, ew.reshape(-1, 1), p['mlp_w1'], p['mlp_b1'].reshape(1, D),
      p['mlp_w2'], p['mlp_b2'].reshape(1, D))


def _mha_body(x_ref, g_ref, b_ref, wq_ref, bq_ref, wk_ref, bk_ref, wv_ref,
              bv_ref, wo_ref, bo_ref, lin1_ref, m_ref, h_ref):
    x = _ln(x_ref[...], g_ref[...], b_ref[...])
    q = jnp.dot(x, wq_ref[...], preferred_element_type=jnp.float32) + bq_ref[...]
    k = jnp.dot(x, wk_ref[...], preferred_element_type=jnp.float32) + bk_ref[...]
    v = jnp.dot(x, wv_ref[...], preferred_element_type=jnp.float32) + bv_ref[...]
    outs = []
    for h in range(NHEAD):
        sl = slice(h * DH, (h + 1) * DH)
        qh, kh, vh = q[:, sl], k[:, sl], v[:, sl]
        att = lax.dot_general(qh, kh, (((1,), (1,)), ((), ())),
                              preferred_element_type=jnp.float32)
        att = jax.nn.softmax(att * (1.0 / math.sqrt(DH)), axis=-1)
        outs.append(jnp.dot(att, vh, preferred_element_type=jnp.float32))
    o = jnp.concatenate(outs, axis=-1)
    m = jnp.dot(o, wo_ref[...], preferred_element_type=jnp.float32) + bo_ref[...]
    m_ref[...] = m
    h_ref[...] = jnp.dot(m, lin1_ref[...], preferred_element_type=jnp.float32)


def _mha(m_x, pm, pmha, lin1_w, seq=512):
    n = m_x.shape[0]
    grid = n // seq
    vec = lambda a: a.reshape(1, D)
    full = pl.BlockSpec((D, D), lambda i: (0, 0))
    vspec = pl.BlockSpec((1, D), lambda i: (0, 0))
    return pl.pallas_call(
        _mha_body,
        grid=(grid,),
        in_specs=[pl.BlockSpec((seq, D), lambda i: (i, 0)),
                  vspec, vspec,
                  full, vspec, full, vspec, full, vspec, full, vspec, full],
        out_specs=[pl.BlockSpec((seq, D), lambda i: (i, 0)),
                   pl.BlockSpec((seq, D), lambda i: (i, 0))],
        out_shape=[jax.ShapeDtypeStruct((n, D), jnp.float32),
                   jax.ShapeDtypeStruct((n, D), jnp.float32)],
    )(m_x, vec(pm['g']), vec(pm['b']),
      pmha['wq'], vec(pmha['bq']), pmha['wk'], vec(pmha['bk']),
      pmha['wv'], vec(pmha['bv']), pmha['wo'], vec(pmha['bo']), lin1_w)


def _post_a_body(part_ref, w2_ref, b2_ref, lw_ref, lb_ref, lin1_ref,
                 a_ref, h_ref):
    agg = part_ref[0] + part_ref[1]
    h2 = _ssp(jnp.dot(agg, w2_ref[...], preferred_element_type=jnp.float32)
              + b2_ref[...])
    a = jnp.dot(h2, lw_ref[...], preferred_element_type=jnp.float32) + lb_ref[...]
    a_ref[...] = a
    h_ref[...] = jnp.dot(a, lin1_ref[...], preferred_element_type=jnp.float32)


def _post_a(part, p, lin1_next, n, block=1000):
    grid = n // block
    vspec = pl.BlockSpec((1, D), lambda i: (0, 0))
    full = pl.BlockSpec((D, D), lambda i: (0, 0))
    return pl.pallas_call(
        _post_a_body,
        grid=(grid,),
        in_specs=[pl.BlockSpec((2, block, D), lambda i: (0, i, 0)),
                  full, vspec, full, vspec, full],
        out_specs=[pl.BlockSpec((block, D), lambda i: (i, 0)),
                   pl.BlockSpec((block, D), lambda i: (i, 0))],
        out_shape=[jax.ShapeDtypeStruct((n, D), jnp.float32),
                   jax.ShapeDtypeStruct((n, D), jnp.float32)],
    )(part, p['lin2_w'], p['lin2_b'].reshape(1, D),
      p['lin_w'], p['lin_b'].reshape(1, D), lin1_next)


def _final_body(part_ref, w2_ref, b2_ref, lw_ref, lb_ref, g_ref, bn_ref,
                base_ref, delta_ref, o_ref):
    agg = part_ref[0] + part_ref[1]
    h2 = _ssp(jnp.dot(agg, w2_ref[...], preferred_element_type=jnp.float32)
              + b2_ref[...])
    msg = jnp.dot(h2, lw_ref[...], preferred_element_type=jnp.float32) + lb_ref[...]
    msg = _ln(msg, g_ref[...], bn_ref[...])
    o_ref[...] = base_ref[...] + msg + delta_ref[...]


def _final(part, p, pln, base, delta, n, block):
    grid = n // block
    vspec = pl.BlockSpec((1, D), lambda i: (0, 0))
    full = pl.BlockSpec((D, D), lambda i: (0, 0))
    rows = pl.BlockSpec((block, D), lambda i: (i, 0))
    return pl.pallas_call(
        _final_body,
        grid=(grid,),
        in_specs=[pl.BlockSpec((2, block, D), lambda i: (0, i, 0)),
                  full, vspec, full, vspec, vspec, vspec, rows, rows],
        out_specs=rows,
        out_shape=jax.ShapeDtypeStruct((n, D), jnp.float32),
    )(part, p['lin2_w'], p['lin2_b'].reshape(1, D),
      p['lin_w'], p['lin_b'].reshape(1, D),
      pln['g'].reshape(1, D), pln['b'].reshape(1, D), base, delta)


def _sc_agg(h, w, src, dst, zeros, n_out, e_pad):
    epw = e_pad // NW
    nchunks = epw // CHUNK
    n_pad = -(-n_out // 128) * 128
    rows_per = n_pad // 16

    mesh = plsc.VectorSubcoreMesh(core_axis_name="c", subcore_axis_name="s")

    @functools.partial(
        pl.kernel,
        out_type=jax.ShapeDtypeStruct((2, n_pad, D), jnp.float32),
        mesh=mesh,
        scratch_types=[
            pltpu.VMEM_SHARED((n_pad, D), jnp.float32),
            pltpu.VMEM((CHUNK,), jnp.int32),
            pltpu.VMEM((CHUNK,), jnp.int32),
            pltpu.VMEM((CHUNK, D), jnp.float32),
            pltpu.VMEM((CHUNK, D), jnp.float32),
            pltpu.SemaphoreType.DMA,
        ],
    )
    def body(h_hbm, w_hbm, src_hbm, dst_hbm, z_hbm, out_hbm,
             acc_sh, src_v, dst_v, rows_v, wv_v, sem):
        c = lax.axis_index("c")
        s = lax.axis_index("s")
        wid = s * 2 + c
        base0 = wid * epw

        @pl.when(s == 0)
        def _():
            pltpu.sync_copy(z_hbm, acc_sh)

        plsc.subcore_barrier()

        def chunk_body(i, carry):
            base = base0 + i * CHUNK
            pltpu.sync_copy(src_hbm.at[pl.ds(base, CHUNK)], src_v)
            pltpu.sync_copy(dst_hbm.at[pl.ds(base, CHUNK)], dst_v)
            gat = pltpu.async_copy(h_hbm.at[src_v], rows_v, sem)
            pltpu.sync_copy(w_hbm.at[pl.ds(base, CHUNK)], wv_v)
            gat.wait()

            def mul_row(e, carry2):
                for j in range(D // 16):
                    sl = pl.ds(j * 16, 16)
                    rows_v[e, sl] = rows_v[e, sl] * wv_v[e, sl]
                return carry2

            lax.fori_loop(0, CHUNK, mul_row, 0)
            pltpu.sync_copy(rows_v, acc_sh.at[dst_v], add=True)
            return carry

        lax.fori_loop(0, nchunks, chunk_body, 0)
        plsc.subcore_barrier()
        pltpu.sync_copy(acc_sh.at[pl.ds(s * rows_per, rows_per)],
                        out_hbm.at[c, pl.ds(s * rows_per, rows_per)])

    return body(h, w, src, dst, zeros)


def _pad_edges(attr, ew, idx):
    e = attr.shape[0]
    e_pad = -(-e // (NW * CHUNK)) * (NW * CHUNK)
    pad = e_pad - e
    attr_p = jnp.pad(attr, ((0, pad), (0, 0)))
    ew_p = jnp.pad(ew, (0, pad))
    src_p = jnp.pad(idx[0], (0, pad))
    dst_p = jnp.pad(idx[1], (0, pad))
    return attr_p, ew_p, src_p, dst_p, e_pad, e


def kernel(a_x, m_x, a2a_edge_index, a2m_edge_index, m2a_edge_index,
           a2a_edge_weights, a2m_edge_weights, m2a_edge_weights,
           a2a_edge_attr, a2m_edge_attr, m2a_edge_attr, params):
    n_a = a_x.shape[0]
    n_m = m_x.shape[0]
    n_a_pad = -(-n_a // 128) * 128
    zeros_a = jnp.zeros((n_a_pad, D), jnp.float32)
    zeros_m = zeros_a[:n_m]

    aa_attr, aa_ew, aa_src, aa_dst, aa_pad, aa_e = _pad_edges(
        a2a_edge_attr, a2a_edge_weights, a2a_edge_index)
    am_attr, am_ew, am_src, am_dst, am_pad, am_e = _pad_edges(
        a2m_edge_attr, a2m_edge_weights, a2m_edge_index)
    ma_attr, ma_ew, ma_src, ma_dst, ma_pad, ma_e = _pad_edges(
        m2a_edge_attr, m2a_edge_weights, m2a_edge_index)

    w_aa = _edge_w(aa_attr, aa_ew, params['short'], aa_pad, aa_e)
    w_am = _edge_w(am_attr, am_ew, params['a2m'], am_pad, am_e)
    w_ma = _edge_w(ma_attr, ma_ew, params['m2a'], ma_pad, ma_e)

    h_short = _pre_a(a_x, params['ln_short']['g'], params['ln_short']['b'],
                     params['short']['lin1_w'])

    m, h_m2a = _mha(m_x, params['ln_long'], params['mha'],
                    params['m2a']['lin1_w'])

    part_aa = _sc_agg(h_short, w_aa, aa_src, aa_dst, zeros_a, n_a, aa_pad)

    a, h_a2m = _post_a(part_aa, params['short'], params['a2m']['lin1_w'], n_a)

    part_am = _sc_agg(h_a2m, w_am, am_src, am_dst, zeros_m, n_m, am_pad)
    part_ma = _sc_agg(h_m2a, w_ma, ma_src, ma_dst, zeros_a, n_a, ma_pad)

    a_out = _final(part_ma, params['m2a'], params['ln_m2a'], a, a_x, n_a, 1000)
    m_out = _final(part_am, params['a2m'], params['ln_a2m'], m, m_x, n_m, 512)
    return (a_out, m_out)


# speedup vs baseline: 2.4588x; 1.6070x over previous
import functools
import math

import jax
import jax.numpy as jnp
from jax import lax
from jax.experimental import pallas as pl
from jax.experimental.pallas import tpu as pltpu
from jax.experimental.pallas import tpu_sc as plsc

D = 128
R = 50
NHEAD = 8
DH = D // NHEAD
LOG2 = math.log(2.0)
EPS = 1e-5

NW = 32
CHUNK = 128
BE = 4096


def _ssp(x):
    return jnp.maximum(x, 0.0) + jnp.log(1.0 + jnp.exp(-jnp.abs(x))) - LOG2


def _ln(x, g, b):
    mu = jnp.mean(x, axis=-1, keepdims=True)
    var = jnp.mean((x - mu) ** 2, axis=-1, keepdims=True)
    return (x - mu) * lax.rsqrt(var + EPS) * g + b


def _pre_a_body(x_ref, g_ref, b_ref, w_ref, o_ref):
    x = _ln(x_ref[...], g_ref[...], b_ref[...])
    o_ref[...] = jnp.dot(x, w_ref[...], preferred_element_type=jnp.float32)


def _pre_a(x, g, b, w, block=1000):
    n = x.shape[0]
    grid = n // block
    return pl.pallas_call(
        _pre_a_body,
        grid=(grid,),
        in_specs=[
            pl.BlockSpec((block, D), lambda i: (i, 0)),
            pl.BlockSpec((1, D), lambda i: (0, 0)),
            pl.BlockSpec((1, D), lambda i: (0, 0)),
            pl.BlockSpec((D, D), lambda i: (0, 0)),
        ],
        out_specs=pl.BlockSpec((block, D), lambda i: (i, 0)),
        out_shape=jax.ShapeDtypeStruct((n, D), jnp.float32),
    )(x, g.reshape(1, D), b.reshape(1, D), w)


def _edge_c_body(ew_ref, o_ref):
    o_ref[...] = 0.5 * (jnp.cos(ew_ref[...] * jnp.pi) + 1.0)


def _edge_c(ew):
    rows = ew.shape[0] // 128
    ew2 = ew.reshape(rows, 128)
    out = pl.pallas_call(
        _edge_c_body,
        grid=(1,),
        in_specs=[pl.BlockSpec((rows, 128), lambda i: (0, 0))],
        out_specs=pl.BlockSpec((rows, 128), lambda i: (0, 0)),
        out_shape=jax.ShapeDtypeStruct((rows, 128), jnp.float32),
    )(ew2)
    return out.reshape(-1, 1)


def _edge_w_body(n_valid, attr_ref, c_ref, w1_ref, b1_ref, w2_ref,
                 b2_ref, o_ref):
    i = pl.program_id(0)
    t = _ssp(jnp.dot(attr_ref[...], w1_ref[...],
                     preferred_element_type=jnp.float32) + b1_ref[...])
    w = jnp.dot(t, w2_ref[...], preferred_element_type=jnp.float32) + b2_ref[...]
    w = w * c_ref[...]
    row = lax.broadcasted_iota(jnp.int32, (BE, 1), 0) + i * BE
    o_ref[...] = jnp.where(row < n_valid, w, 0.0)


def _edge_w(attr, c, p, e_pad):
    n_valid = attr.shape[0]
    grid = e_pad // BE
    return pl.pallas_call(
        functools.partial(_edge_w_body, n_valid),
        grid=(grid,),
        in_specs=[
            pl.BlockSpec((BE, R), lambda i: (i, 0)),
            pl.BlockSpec((BE, 1), lambda i: (i, 0)),
            pl.BlockSpec((R, D), lambda i: (0, 0)),
            pl.BlockSpec((1, D), lambda i: (0, 0)),
            pl.BlockSpec((D, D), lambda i: (0, 0)),
            pl.BlockSpec((1, D), lambda i: (0, 0)),
        ],
        out_specs=pl.BlockSpec((BE, D), lambda i: (i, 0)),
        out_shape=jax.ShapeDtypeStruct((e_pad, D), jnp.float32),
    )(attr, c, p['mlp_w1'], p['mlp_b1'].reshape(1, D),
      p['mlp_w2'], p['mlp_b2'].reshape(1, D))


def _mha_body(x_ref, g_ref, b_ref, wq_ref, bq_ref, wk_ref, bk_ref, wv_ref,
              bv_ref, wo_ref, bo_ref, lin1_ref, m_ref, h_ref):
    x = _ln(x_ref[...], g_ref[...], b_ref[...])
    q = jnp.dot(x, wq_ref[...], preferred_element_type=jnp.float32) + bq_ref[...]
    k = jnp.dot(x, wk_ref[...], preferred_element_type=jnp.float32) + bk_ref[...]
    v = jnp.dot(x, wv_ref[...], preferred_element_type=jnp.float32) + bv_ref[...]
    outs = []
    for h in range(NHEAD):
        sl = slice(h * DH, (h + 1) * DH)
        qh, kh, vh = q[:, sl], k[:, sl], v[:, sl]
        att = lax.dot_general(qh, kh, (((1,), (1,)), ((), ())),
                              preferred_element_type=jnp.float32)
        att = jax.nn.softmax(att * (1.0 / math.sqrt(DH)), axis=-1)
        outs.append(jnp.dot(att, vh, preferred_element_type=jnp.float32))
    o = jnp.concatenate(outs, axis=-1)
    m = jnp.dot(o, wo_ref[...], preferred_element_type=jnp.float32) + bo_ref[...]
    m_ref[...] = m
    h_ref[...] = jnp.dot(m, lin1_ref[...], preferred_element_type=jnp.float32)


def _mha(m_x, pm, pmha, lin1_w, seq=512):
    n = m_x.shape[0]
    grid = n // seq
    vec = lambda a: a.reshape(1, D)
    full = pl.BlockSpec((D, D), lambda i: (0, 0))
    vspec = pl.BlockSpec((1, D), lambda i: (0, 0))
    return pl.pallas_call(
        _mha_body,
        grid=(grid,),
        in_specs=[pl.BlockSpec((seq, D), lambda i: (i, 0)),
                  vspec, vspec,
                  full, vspec, full, vspec, full, vspec, full, vspec, full],
        out_specs=[pl.BlockSpec((seq, D), lambda i: (i, 0)),
                   pl.BlockSpec((seq, D), lambda i: (i, 0))],
        out_shape=[jax.ShapeDtypeStruct((n, D), jnp.float32),
                   jax.ShapeDtypeStruct((n, D), jnp.float32)],
    )(m_x, vec(pm['g']), vec(pm['b']),
      pmha['wq'], vec(pmha['bq']), pmha['wk'], vec(pmha['bk']),
      pmha['wv'], vec(pmha['bv']), pmha['wo'], vec(pmha['bo']), lin1_w)


def _post_a_body(part_ref, w2_ref, b2_ref, lw_ref, lb_ref, lin1_ref,
                 a_ref, h_ref):
    agg = part_ref[0] + part_ref[1]
    h2 = _ssp(jnp.dot(agg, w2_ref[...], preferred_element_type=jnp.float32)
              + b2_ref[...])
    a = jnp.dot(h2, lw_ref[...], preferred_element_type=jnp.float32) + lb_ref[...]
    a_ref[...] = a
    h_ref[...] = jnp.dot(a, lin1_ref[...], preferred_element_type=jnp.float32)


def _post_a(part, p, lin1_next, n, block=1000):
    grid = n // block
    vspec = pl.BlockSpec((1, D), lambda i: (0, 0))
    full = pl.BlockSpec((D, D), lambda i: (0, 0))
    return pl.pallas_call(
        _post_a_body,
        grid=(grid,),
        in_specs=[pl.BlockSpec((2, block, D), lambda i: (0, i, 0)),
                  full, vspec, full, vspec, full],
        out_specs=[pl.BlockSpec((block, D), lambda i: (i, 0)),
                   pl.BlockSpec((block, D), lambda i: (i, 0))],
        out_shape=[jax.ShapeDtypeStruct((n, D), jnp.float32),
                   jax.ShapeDtypeStruct((n, D), jnp.float32)],
    )(part, p['lin2_w'], p['lin2_b'].reshape(1, D),
      p['lin_w'], p['lin_b'].reshape(1, D), lin1_next)


def _final_body(n_valid, block, part_ref, w2_ref, b2_ref, lw_ref, lb_ref,
                g_ref, bn_ref, base_ref, delta_ref, o_ref):
    i = pl.program_id(0)
    row = lax.broadcasted_iota(jnp.int32, (block, 1), 0) + i * block
    agg = jnp.where(row < n_valid, part_ref[0] + part_ref[1], 0.0)
    h2 = _ssp(jnp.dot(agg, w2_ref[...], preferred_element_type=jnp.float32)
              + b2_ref[...])
    msg = jnp.dot(h2, lw_ref[...], preferred_element_type=jnp.float32) + lb_ref[...]
    msg = _ln(msg, g_ref[...], bn_ref[...])
    o_ref[...] = base_ref[...] + msg + delta_ref[...]


def _final(part, p, pln, base, delta, n, block):
    grid = n // block
    n_valid = part.shape[1]
    maxblk = (n_valid - 1) // block
    vspec = pl.BlockSpec((1, D), lambda i: (0, 0))
    full = pl.BlockSpec((D, D), lambda i: (0, 0))
    rows = pl.BlockSpec((block, D), lambda i: (i, 0))
    return pl.pallas_call(
        functools.partial(_final_body, n_valid, block),
        grid=(grid,),
        in_specs=[pl.BlockSpec((2, block, D),
                               lambda i: (0, jnp.minimum(i, maxblk), 0)),
                  full, vspec, full, vspec, vspec, vspec, rows, rows],
        out_specs=rows,
        out_shape=jax.ShapeDtypeStruct((n, D), jnp.float32),
    )(part, p['lin2_w'], p['lin2_b'].reshape(1, D),
      p['lin_w'], p['lin_b'].reshape(1, D),
      pln['g'].reshape(1, D), pln['b'].reshape(1, D), base, delta)


def _sc_agg(h, w, src, dst, z128, n_out, e_pad):
    chunk = 128 if n_out <= 2048 else 64
    epw = e_pad // NW
    nchunks = epw // chunk
    n_pad = -(-n_out // 128) * 128
    rows_per = n_pad // 16

    mesh = plsc.VectorSubcoreMesh(core_axis_name="c", subcore_axis_name="s")

    @functools.partial(
        pl.kernel,
        out_type=jax.ShapeDtypeStruct((2, n_pad, D), jnp.float32),
        mesh=mesh,
        scratch_types=[
            pltpu.VMEM_SHARED((n_pad, D), jnp.float32),
            pltpu.VMEM((chunk,), jnp.int32),
            pltpu.VMEM((chunk,), jnp.int32),
            pltpu.VMEM((chunk,), jnp.int32),
            pltpu.VMEM((chunk,), jnp.int32),
            pltpu.VMEM((chunk, D), jnp.float32),
            pltpu.VMEM((chunk, D), jnp.float32),
            pltpu.VMEM((chunk, D), jnp.float32),
            pltpu.VMEM((chunk, D), jnp.float32),
            pltpu.SemaphoreType.DMA,
            pltpu.SemaphoreType.DMA,
            pltpu.SemaphoreType.DMA,
            pltpu.SemaphoreType.DMA,
            pltpu.SemaphoreType.DMA,
            pltpu.SemaphoreType.DMA,
        ],
    )
    def body(h_hbm, w_hbm, src_hbm, dst_hbm, z_hbm, out_hbm,
             acc_sh, src0, src1, dst0, dst1, rows0, rows1, wv0, wv1,
             semi0, semi1, semg0, semg1, semw0, semw1):
        c = lax.axis_index("c")
        s = lax.axis_index("s")
        wid = s * 2 + c
        srcb = (src0, src1)
        dstb = (dst0, dst1)
        rowsb = (rows0, rows1)
        wvb = (wv0, wv1)
        semi = (semi0, semi1)
        semg = (semg0, semg1)
        semw = (semw0, semw1)

        r0 = s * rows_per
        off = 0
        while off < rows_per:
            nrow = min(128, rows_per - off)
            pltpu.sync_copy(z_hbm.at[pl.ds(0, nrow)],
                            acc_sh.at[pl.ds(r0 + off, nrow)])
            off += nrow
        plsc.subcore_barrier()

        ebase = wid * epw

        def issue_idx(k, b):
            pltpu.async_copy(src_hbm.at[pl.ds(ebase + k * chunk, chunk)],
                             srcb[b], semi[b])
            pltpu.async_copy(dst_hbm.at[pl.ds(ebase + k * chunk, chunk)],
                             dstb[b], semi[b])

        def wait_idx(k, b):
            pltpu.make_async_copy(src_hbm.at[pl.ds(ebase + k * chunk, chunk)],
                                  srcb[b], semi[b]).wait()
            pltpu.make_async_copy(dst_hbm.at[pl.ds(ebase + k * chunk, chunk)],
                                  dstb[b], semi[b]).wait()

        def issue_gw(k, b):
            pltpu.async_copy(h_hbm.at[srcb[b]], rowsb[b], semg[b])
            pltpu.async_copy(w_hbm.at[pl.ds(ebase + k * chunk, chunk)],
                             wvb[b], semw[b])

        def consume(k, b):
            pltpu.make_async_copy(h_hbm.at[srcb[b]], rowsb[b], semg[b]).wait()
            pltpu.make_async_copy(w_hbm.at[pl.ds(ebase + k * chunk, chunk)],
                                  wvb[b], semw[b]).wait()

            @plsc.parallel_loop(0, chunk, unroll=4)
            def _(e):
                for j in range(D // 16):
                    sl = pl.ds(j * 16, 16)
                    rowsb[b][e, sl] = rowsb[b][e, sl] * wvb[b][e, sl]

            pltpu.sync_copy(rowsb[b], acc_sh.at[dstb[b]], add=True)

        pltpu.sync_copy(src_hbm.at[pl.ds(ebase, chunk)], src0)
        pltpu.sync_copy(dst_hbm.at[pl.ds(ebase, chunk)], dst0)
        issue_gw(0, 0)
        issue_idx(1, 1)

        def step(k, b):
            bn = 1 - b

            @pl.when(k + 1 < nchunks)
            def _():
                wait_idx(k + 1, bn)
                issue_gw(k + 1, bn)

            consume(k, b)

            @pl.when(k + 2 < nchunks)
            def _():
                issue_idx(k + 2, b)

        def pair(j, carry):
            step(2 * j, 0)
            step(2 * j + 1, 1)
            return carry

        lax.fori_loop(0, nchunks // 2, pair, 0)
        if nchunks % 2:
            step(nchunks - 1, 0)

        plsc.subcore_barrier()
        pltpu.sync_copy(acc_sh.at[pl.ds(s * rows_per, rows_per)],
                        out_hbm.at[c, pl.ds(s * rows_per, rows_per)])

    return body(h, w, src, dst, z128)


def _pad_idx(idx, e_pad):
    e = idx.shape[1]
    src = jnp.pad(idx[0], (0, e_pad - e))
    dst = jnp.pad(idx[1], (0, e_pad - e))
    return src, dst


def kernel(a_x, m_x, a2a_edge_index, a2m_edge_index, m2a_edge_index,
           a2a_edge_weights, a2m_edge_weights, m2a_edge_weights,
           a2a_edge_attr, a2m_edge_attr, m2a_edge_attr, params):
    n_a = a_x.shape[0]
    n_m = m_x.shape[0]
    z128 = jnp.zeros((128, D), jnp.float32)

    aa_pad = -(-a2a_edge_attr.shape[0] // (NW * CHUNK)) * (NW * CHUNK)
    am_pad = -(-a2m_edge_attr.shape[0] // (NW * CHUNK)) * (NW * CHUNK)
    ma_pad = -(-m2a_edge_attr.shape[0] // (NW * CHUNK)) * (NW * CHUNK)
    aa_src, aa_dst = _pad_idx(a2a_edge_index, aa_pad)
    am_src, am_dst = _pad_idx(a2m_edge_index, am_pad)
    ma_src, ma_dst = _pad_idx(m2a_edge_index, ma_pad)

    w_aa = _edge_w(a2a_edge_attr, _edge_c(a2a_edge_weights),
                   params['short'], aa_pad)
    w_am = _edge_w(a2m_edge_attr, _edge_c(a2m_edge_weights),
                   params['a2m'], am_pad)
    w_ma = _edge_w(m2a_edge_attr, _edge_c(m2a_edge_weights),
                   params['m2a'], ma_pad)

    h_short = _pre_a(a_x, params['ln_short']['g'], params['ln_short']['b'],
                     params['short']['lin1_w'])

    m, h_m2a = _mha(m_x, params['ln_long'], params['mha'],
                    params['m2a']['lin1_w'])

    part_aa = _sc_agg(h_short, w_aa, aa_src, aa_dst, z128, n_a, aa_pad)

    a, h_a2m = _post_a(part_aa, params['short'], params['a2m']['lin1_w'], n_a)

    part_am = _sc_agg(h_a2m, w_am, am_src, am_dst, z128, n_m, am_pad)
    part_ma = _sc_agg(h_m2a, w_ma, ma_src, ma_dst, z128, n_m, ma_pad)

    a_out = _final(part_ma, params['m2a'], params['ln_m2a'], a, a_x, n_a, 1000)
    m_out = _final(part_am, params['a2m'], params['ln_a2m'], m, m_x, n_m, 512)
    return (a_out, m_out)


# speedup vs baseline: 2.9231x; 1.1888x over previous
import functools
import math

import jax
import jax.numpy as jnp
from jax import lax
from jax.experimental import pallas as pl
from jax.experimental.pallas import tpu as pltpu
from jax.experimental.pallas import tpu_sc as plsc

D = 128
R = 50
NHEAD = 8
DH = D // NHEAD
LOG2 = math.log(2.0)
EPS = 1e-5

NW = 32
CHUNK = 128
BE = 4096


def _ssp(x):
    return jnp.maximum(x, 0.0) + jnp.log(1.0 + jnp.exp(-jnp.abs(x))) - LOG2


def _ln(x, g, b):
    mu = jnp.mean(x, axis=-1, keepdims=True)
    var = jnp.mean((x - mu) ** 2, axis=-1, keepdims=True)
    return (x - mu) * lax.rsqrt(var + EPS) * g + b


def _pre_a_body(x_ref, g_ref, b_ref, w_ref, o_ref):
    x = _ln(x_ref[...], g_ref[...], b_ref[...])
    o_ref[...] = jnp.dot(x, w_ref[...], preferred_element_type=jnp.float32)


def _pre_a(x, g, b, w, block=1000):
    n = x.shape[0]
    grid = n // block
    return pl.pallas_call(
        _pre_a_body,
        grid=(grid,),
        in_specs=[
            pl.BlockSpec((block, D), lambda i: (i, 0)),
            pl.BlockSpec((1, D), lambda i: (0, 0)),
            pl.BlockSpec((1, D), lambda i: (0, 0)),
            pl.BlockSpec((D, D), lambda i: (0, 0)),
        ],
        out_specs=pl.BlockSpec((block, D), lambda i: (i, 0)),
        out_shape=jax.ShapeDtypeStruct((n, D), jnp.float32),
    )(x, g.reshape(1, D), b.reshape(1, D), w)


def _edge_w_body(n_valid, attr_ref, ew_ref, w1_ref, b1_ref, w2_ref,
                 b2_ref, o_ref):
    i = pl.program_id(0)
    t = _ssp(jnp.dot(attr_ref[...], w1_ref[...],
                     preferred_element_type=jnp.float32) + b1_ref[...])
    w = jnp.dot(t, w2_ref[...], preferred_element_type=jnp.float32) + b2_ref[...]
    cblk = 0.5 * (jnp.cos(ew_ref[...] * jnp.pi) + 1.0)
    w = (w.reshape(BE // 128, 128, D) * cblk[:, :, None]).reshape(BE, D)
    row = lax.broadcasted_iota(jnp.int32, (BE, 1), 0) + i * BE
    o_ref[...] = jnp.where(row < n_valid, w, 0.0)


def _edge_w(attr, ew, p, e_pad):
    n_valid = attr.shape[0]
    grid = e_pad // BE
    ew2 = ew.reshape(-1, 128)
    return pl.pallas_call(
        functools.partial(_edge_w_body, n_valid),
        grid=(grid,),
        in_specs=[
            pl.BlockSpec((BE, R), lambda i: (i, 0)),
            pl.BlockSpec((BE // 128, 128), lambda i: (i, 0)),
            pl.BlockSpec((R, D), lambda i: (0, 0)),
            pl.BlockSpec((1, D), lambda i: (0, 0)),
            pl.BlockSpec((D, D), lambda i: (0, 0)),
            pl.BlockSpec((1, D), lambda i: (0, 0)),
        ],
        out_specs=pl.BlockSpec((BE, D), lambda i: (i, 0)),
        out_shape=jax.ShapeDtypeStruct((e_pad, D), jnp.float32),
    )(attr, ew2, p['mlp_w1'], p['mlp_b1'].reshape(1, D),
      p['mlp_w2'], p['mlp_b2'].reshape(1, D))


def _mha_body(x_ref, g_ref, b_ref, wq_ref, bq_ref, wk_ref, bk_ref, wv_ref,
              bv_ref, wo_ref, bo_ref, lin1_ref, m_ref, h_ref):
    x = _ln(x_ref[...], g_ref[...], b_ref[...])
    q = jnp.dot(x, wq_ref[...], preferred_element_type=jnp.float32) + bq_ref[...]
    k = jnp.dot(x, wk_ref[...], preferred_element_type=jnp.float32) + bk_ref[...]
    v = jnp.dot(x, wv_ref[...], preferred_element_type=jnp.float32) + bv_ref[...]
    outs = []
    for h in range(NHEAD):
        sl = slice(h * DH, (h + 1) * DH)
        qh, kh, vh = q[:, sl], k[:, sl], v[:, sl]
        att = lax.dot_general(qh, kh, (((1,), (1,)), ((), ())),
                              preferred_element_type=jnp.float32)
        att = jax.nn.softmax(att * (1.0 / math.sqrt(DH)), axis=-1)
        outs.append(jnp.dot(att, vh, preferred_element_type=jnp.float32))
    o = jnp.concatenate(outs, axis=-1)
    m = jnp.dot(o, wo_ref[...], preferred_element_type=jnp.float32) + bo_ref[...]
    m_ref[...] = m
    h_ref[...] = jnp.dot(m, lin1_ref[...], preferred_element_type=jnp.float32)


def _mha(m_x, pm, pmha, lin1_w, seq=512):
    n = m_x.shape[0]
    grid = n // seq
    vec = lambda a: a.reshape(1, D)
    full = pl.BlockSpec((D, D), lambda i: (0, 0))
    vspec = pl.BlockSpec((1, D), lambda i: (0, 0))
    return pl.pallas_call(
        _mha_body,
        grid=(grid,),
        in_specs=[pl.BlockSpec((seq, D), lambda i: (i, 0)),
                  vspec, vspec,
                  full, vspec, full, vspec, full, vspec, full, vspec, full],
        out_specs=[pl.BlockSpec((seq, D), lambda i: (i, 0)),
                   pl.BlockSpec((seq, D), lambda i: (i, 0))],
        out_shape=[jax.ShapeDtypeStruct((n, D), jnp.float32),
                   jax.ShapeDtypeStruct((n, D), jnp.float32)],
    )(m_x, vec(pm['g']), vec(pm['b']),
      pmha['wq'], vec(pmha['bq']), pmha['wk'], vec(pmha['bk']),
      pmha['wv'], vec(pmha['bv']), pmha['wo'], vec(pmha['bo']), lin1_w)


def _post_a_body(part_ref, w2_ref, b2_ref, lw_ref, lb_ref, lin1_ref,
                 a_ref, h_ref):
    agg = part_ref[0] + part_ref[1]
    h2 = _ssp(jnp.dot(agg, w2_ref[...], preferred_element_type=jnp.float32)
              + b2_ref[...])
    a = jnp.dot(h2, lw_ref[...], preferred_element_type=jnp.float32) + lb_ref[...]
    a_ref[...] = a
    h_ref[...] = jnp.dot(a, lin1_ref[...], preferred_element_type=jnp.float32)


def _post_a(part, p, lin1_next, n, block=1000):
    grid = n // block
    vspec = pl.BlockSpec((1, D), lambda i: (0, 0))
    full = pl.BlockSpec((D, D), lambda i: (0, 0))
    return pl.pallas_call(
        _post_a_body,
        grid=(grid,),
        in_specs=[pl.BlockSpec((2, block, D), lambda i: (0, i, 0)),
                  full, vspec, full, vspec, full],
        out_specs=[pl.BlockSpec((block, D), lambda i: (i, 0)),
                   pl.BlockSpec((block, D), lambda i: (i, 0))],
        out_shape=[jax.ShapeDtypeStruct((n, D), jnp.float32),
                   jax.ShapeDtypeStruct((n, D), jnp.float32)],
    )(part, p['lin2_w'], p['lin2_b'].reshape(1, D),
      p['lin_w'], p['lin_b'].reshape(1, D), lin1_next)


def _final_body(n_valid, block, part_ref, w2_ref, b2_ref, lw_ref, lb_ref,
                g_ref, bn_ref, base_ref, delta_ref, o_ref):
    i = pl.program_id(0)
    row = lax.broadcasted_iota(jnp.int32, (block, 1), 0) + i * block
    agg = jnp.where(row < n_valid, part_ref[0] + part_ref[1], 0.0)
    h2 = _ssp(jnp.dot(agg, w2_ref[...], preferred_element_type=jnp.float32)
              + b2_ref[...])
    msg = jnp.dot(h2, lw_ref[...], preferred_element_type=jnp.float32) + lb_ref[...]
    msg = _ln(msg, g_ref[...], bn_ref[...])
    o_ref[...] = base_ref[...] + msg + delta_ref[...]


def _final(part, p, pln, base, delta, n, block):
    grid = n // block
    n_valid = part.shape[1]
    maxblk = (n_valid - 1) // block
    vspec = pl.BlockSpec((1, D), lambda i: (0, 0))
    full = pl.BlockSpec((D, D), lambda i: (0, 0))
    rows = pl.BlockSpec((block, D), lambda i: (i, 0))
    return pl.pallas_call(
        functools.partial(_final_body, n_valid, block),
        grid=(grid,),
        in_specs=[pl.BlockSpec((2, block, D),
                               lambda i: (0, jnp.minimum(i, maxblk), 0)),
                  full, vspec, full, vspec, vspec, vspec, rows, rows],
        out_specs=rows,
        out_shape=jax.ShapeDtypeStruct((n, D), jnp.float32),
    )(part, p['lin2_w'], p['lin2_b'].reshape(1, D),
      p['lin_w'], p['lin_b'].reshape(1, D),
      pln['g'].reshape(1, D), pln['b'].reshape(1, D), base, delta)


def _sc_agg(h, w, src, dst, z128, n_out, e_pad):
    chunk = 128 if n_out <= 2048 else 64
    ntot = e_pad // (16 * chunk)
    n0 = int(round(ntot * 0.59 / 2)) * 2
    n1 = ntot - n0
    assert n1 % 2 == 0 and n1 > 2
    n_pad = -(-n_out // 128) * 128
    rows_per = n_pad // 16

    mesh = plsc.VectorSubcoreMesh(core_axis_name="c", subcore_axis_name="s")

    @functools.partial(
        pl.kernel,
        out_type=jax.ShapeDtypeStruct((2, n_pad, D), jnp.float32),
        mesh=mesh,
        scratch_types=[
            pltpu.VMEM_SHARED((n_pad, D), jnp.float32),
            pltpu.VMEM((chunk,), jnp.int32),
            pltpu.VMEM((chunk,), jnp.int32),
            pltpu.VMEM((chunk,), jnp.int32),
            pltpu.VMEM((chunk,), jnp.int32),
            pltpu.VMEM((chunk, D), jnp.float32),
            pltpu.VMEM((chunk, D), jnp.float32),
            pltpu.VMEM((chunk, D), jnp.float32),
            pltpu.VMEM((chunk, D), jnp.float32),
            pltpu.SemaphoreType.DMA,
            pltpu.SemaphoreType.DMA,
            pltpu.SemaphoreType.DMA,
            pltpu.SemaphoreType.DMA,
            pltpu.SemaphoreType.DMA,
            pltpu.SemaphoreType.DMA,
        ],
    )
    def body(h_hbm, w_hbm, src_hbm, dst_hbm, z_hbm, out_hbm,
             acc_sh, src0, src1, dst0, dst1, rows0, rows1, wv0, wv1,
             semi0, semi1, semg0, semg1, semw0, semw1):
        c = lax.axis_index("c")
        s = lax.axis_index("s")
        myn = jnp.where(c == 0, n0, n1)
        srcb = (src0, src1)
        dstb = (dst0, dst1)
        rowsb = (rows0, rows1)
        wvb = (wv0, wv1)
        semi = (semi0, semi1)
        semg = (semg0, semg1)
        semw = (semw0, semw1)

        r0 = s * rows_per
        off = 0
        while off < rows_per:
            nrow = min(128, rows_per - off)
            pltpu.sync_copy(z_hbm.at[pl.ds(0, nrow)],
                            acc_sh.at[pl.ds(r0 + off, nrow)])
            off += nrow
        plsc.subcore_barrier()

        ebase = (s * ntot + c * n0) * chunk

        def issue_idx(k, b):
            pltpu.async_copy(src_hbm.at[pl.ds(ebase + k * chunk, chunk)],
                             srcb[b], semi[b])
            pltpu.async_copy(dst_hbm.at[pl.ds(ebase + k * chunk, chunk)],
                             dstb[b], semi[b])

        def wait_idx(k, b):
            pltpu.make_async_copy(src_hbm.at[pl.ds(ebase + k * chunk, chunk)],
                                  srcb[b], semi[b]).wait()
            pltpu.make_async_copy(dst_hbm.at[pl.ds(ebase + k * chunk, chunk)],
                                  dstb[b], semi[b]).wait()

        def issue_gw(k, b):
            pltpu.async_copy(h_hbm.at[srcb[b]], rowsb[b], semg[b])
            pltpu.async_copy(w_hbm.at[pl.ds(ebase + k * chunk, chunk)],
                             wvb[b], semw[b])

        def consume(k, b):
            pltpu.make_async_copy(h_hbm.at[srcb[b]], rowsb[b], semg[b]).wait()
            pltpu.make_async_copy(w_hbm.at[pl.ds(ebase + k * chunk, chunk)],
                                  wvb[b], semw[b]).wait()

            @plsc.parallel_loop(0, chunk, unroll=4)
            def _(e):
                for j in range(D // 16):
                    sl = pl.ds(j * 16, 16)
                    rowsb[b][e, sl] = rowsb[b][e, sl] * wvb[b][e, sl]

            pltpu.sync_copy(rowsb[b], acc_sh.at[dstb[b]], add=True)

        pltpu.sync_copy(src_hbm.at[pl.ds(ebase, chunk)], src0)
        pltpu.sync_copy(dst_hbm.at[pl.ds(ebase, chunk)], dst0)
        issue_gw(0, 0)
        issue_idx(1, 1)

        def step(k, b):
            bn = 1 - b

            @pl.when(k + 1 < myn)
            def _():
                wait_idx(k + 1, bn)
                issue_gw(k + 1, bn)

            consume(k, b)

            @pl.when(k + 2 < myn)
            def _():
                issue_idx(k + 2, b)

        def pair(j, carry):
            step(2 * j, 0)
            step(2 * j + 1, 1)
            return carry

        lax.fori_loop(0, myn // 2, pair, 0)

        plsc.subcore_barrier()
        pltpu.sync_copy(acc_sh.at[pl.ds(s * rows_per, rows_per)],
                        out_hbm.at[c, pl.ds(s * rows_per, rows_per)])

    return body(h, w, src, dst, z128)


def _pad_idx(idx, e_pad):
    e = idx.shape[1]
    src = jnp.pad(idx[0], (0, e_pad - e))
    dst = jnp.pad(idx[1], (0, e_pad - e))
    return src, dst


def kernel(a_x, m_x, a2a_edge_index, a2m_edge_index, m2a_edge_index,
           a2a_edge_weights, a2m_edge_weights, m2a_edge_weights,
           a2a_edge_attr, a2m_edge_attr, m2a_edge_attr, params):
    n_a = a_x.shape[0]
    n_m = m_x.shape[0]
    z128 = jnp.zeros((128, D), jnp.float32)

    aa_pad = -(-a2a_edge_attr.shape[0] // (NW * CHUNK)) * (NW * CHUNK)
    am_pad = -(-a2m_edge_attr.shape[0] // (NW * CHUNK)) * (NW * CHUNK)
    ma_pad = -(-m2a_edge_attr.shape[0] // (NW * CHUNK)) * (NW * CHUNK)
    aa_src, aa_dst = _pad_idx(a2a_edge_index, aa_pad)
    am_src, am_dst = _pad_idx(a2m_edge_index, am_pad)
    ma_src, ma_dst = _pad_idx(m2a_edge_index, ma_pad)

    w_aa = _edge_w(a2a_edge_attr, a2a_edge_weights, params['short'], aa_pad)
    w_am = _edge_w(a2m_edge_attr, a2m_edge_weights, params['a2m'], am_pad)
    w_ma = _edge_w(m2a_edge_attr, m2a_edge_weights, params['m2a'], ma_pad)

    h_short = _pre_a(a_x, params['ln_short']['g'], params['ln_short']['b'],
                     params['short']['lin1_w'])

    m, h_m2a = _mha(m_x, params['ln_long'], params['mha'],
                    params['m2a']['lin1_w'])

    part_aa = _sc_agg(h_short, w_aa, aa_src, aa_dst, z128, n_a, aa_pad)

    a, h_a2m = _post_a(part_aa, params['short'], params['a2m']['lin1_w'], n_a)

    part_am = _sc_agg(h_a2m, w_am, am_src, am_dst, z128, n_m, am_pad)
    part_ma = _sc_agg(h_m2a, w_ma, ma_src, ma_dst, z128, n_m, ma_pad)

    a_out = _final(part_ma, params['m2a'], params['ln_m2a'], a, a_x, n_a, 1000)
    m_out = _final(part_am, params['a2m'], params['ln_a2m'], m, m_x, n_m, 512)
    return (a_out, m_out)


# speedup vs baseline: 3.0369x; 1.0389x over previous
import functools
import math

import jax
import jax.numpy as jnp
from jax import lax
from jax.experimental import pallas as pl
from jax.experimental.pallas import tpu as pltpu
from jax.experimental.pallas import tpu_sc as plsc

D = 128
R = 50
NHEAD = 8
DH = D // NHEAD
LOG2 = math.log(2.0)
EPS = 1e-5

NW = 32
CHUNK = 128
BE = 4096


def _ssp(x):
    return jnp.maximum(x, 0.0) + jnp.log(1.0 + jnp.exp(-jnp.abs(x))) - LOG2


def _ln(x, g, b):
    mu = jnp.mean(x, axis=-1, keepdims=True)
    var = jnp.mean((x - mu) ** 2, axis=-1, keepdims=True)
    return (x - mu) * lax.rsqrt(var + EPS) * g + b


def _pre_a_body(x_ref, g_ref, b_ref, w_ref, o_ref):
    x = _ln(x_ref[...], g_ref[...], b_ref[...])
    o_ref[...] = jnp.dot(x, w_ref[...], preferred_element_type=jnp.float32)


def _pre_a(x, g, b, w, block=1000):
    n = x.shape[0]
    grid = n // block
    return pl.pallas_call(
        _pre_a_body,
        grid=(grid,),
        in_specs=[
            pl.BlockSpec((block, D), lambda i: (i, 0)),
            pl.BlockSpec((1, D), lambda i: (0, 0)),
            pl.BlockSpec((1, D), lambda i: (0, 0)),
            pl.BlockSpec((D, D), lambda i: (0, 0)),
        ],
        out_specs=pl.BlockSpec((block, D), lambda i: (i, 0)),
        out_shape=jax.ShapeDtypeStruct((n, D), jnp.float32),
    )(x, g.reshape(1, D), b.reshape(1, D), w)


def _edge_w_body(n_valid, attr_ref, ew_ref, w1_ref, b1_ref, w2_ref,
                 b2_ref, o_ref):
    i = pl.program_id(0)
    t = _ssp(jnp.dot(attr_ref[...], w1_ref[...],
                     preferred_element_type=jnp.float32) + b1_ref[...])
    w = jnp.dot(t, w2_ref[...], preferred_element_type=jnp.float32) + b2_ref[...]
    cblk = 0.5 * (jnp.cos(ew_ref[...] * jnp.pi) + 1.0)
    w = (w.reshape(BE // 128, 128, D) * cblk[:, :, None]).reshape(BE, D)
    row = lax.broadcasted_iota(jnp.int32, (BE, 1), 0) + i * BE
    o_ref[...] = jnp.where(row < n_valid, w, 0.0)


def _edge_w(attr, ew, p, e_pad):
    n_valid = attr.shape[0]
    grid = e_pad // BE
    ew2 = ew.reshape(-1, 128)
    return pl.pallas_call(
        functools.partial(_edge_w_body, n_valid),
        grid=(grid,),
        in_specs=[
            pl.BlockSpec((BE, R), lambda i: (i, 0)),
            pl.BlockSpec((BE // 128, 128), lambda i: (i, 0)),
            pl.BlockSpec((R, D), lambda i: (0, 0)),
            pl.BlockSpec((1, D), lambda i: (0, 0)),
            pl.BlockSpec((D, D), lambda i: (0, 0)),
            pl.BlockSpec((1, D), lambda i: (0, 0)),
        ],
        out_specs=pl.BlockSpec((BE, D), lambda i: (i, 0)),
        out_shape=jax.ShapeDtypeStruct((e_pad, D), jnp.float32),
    )(attr, ew2, p['mlp_w1'], p['mlp_b1'].reshape(1, D),
      p['mlp_w2'], p['mlp_b2'].reshape(1, D))


def _mha_body(x_ref, g_ref, b_ref, wq_ref, bq_ref, wk_ref, bk_ref, wv_ref,
              bv_ref, wo_ref, bo_ref, lin1_ref, m_ref, h_ref):
    x = _ln(x_ref[...], g_ref[...], b_ref[...])
    q = jnp.dot(x, wq_ref[...], preferred_element_type=jnp.float32) + bq_ref[...]
    k = jnp.dot(x, wk_ref[...], preferred_element_type=jnp.float32) + bk_ref[...]
    v = jnp.dot(x, wv_ref[...], preferred_element_type=jnp.float32) + bv_ref[...]
    outs = []
    for h in range(NHEAD):
        sl = slice(h * DH, (h + 1) * DH)
        qh, kh, vh = q[:, sl], k[:, sl], v[:, sl]
        att = lax.dot_general(qh, kh, (((1,), (1,)), ((), ())),
                              preferred_element_type=jnp.float32)
        att = jax.nn.softmax(att * (1.0 / math.sqrt(DH)), axis=-1)
        outs.append(jnp.dot(att, vh, preferred_element_type=jnp.float32))
    o = jnp.concatenate(outs, axis=-1)
    m = jnp.dot(o, wo_ref[...], preferred_element_type=jnp.float32) + bo_ref[...]
    m_ref[...] = m
    h_ref[...] = jnp.dot(m, lin1_ref[...], preferred_element_type=jnp.float32)


def _mha(m_x, pm, pmha, lin1_w, seq=512):
    n = m_x.shape[0]
    grid = n // seq
    vec = lambda a: a.reshape(1, D)
    full = pl.BlockSpec((D, D), lambda i: (0, 0))
    vspec = pl.BlockSpec((1, D), lambda i: (0, 0))
    return pl.pallas_call(
        _mha_body,
        grid=(grid,),
        in_specs=[pl.BlockSpec((seq, D), lambda i: (i, 0)),
                  vspec, vspec,
                  full, vspec, full, vspec, full, vspec, full, vspec, full],
        out_specs=[pl.BlockSpec((seq, D), lambda i: (i, 0)),
                   pl.BlockSpec((seq, D), lambda i: (i, 0))],
        out_shape=[jax.ShapeDtypeStruct((n, D), jnp.float32),
                   jax.ShapeDtypeStruct((n, D), jnp.float32)],
    )(m_x, vec(pm['g']), vec(pm['b']),
      pmha['wq'], vec(pmha['bq']), pmha['wk'], vec(pmha['bk']),
      pmha['wv'], vec(pmha['bv']), pmha['wo'], vec(pmha['bo']), lin1_w)


def _post_a_body(part_ref, w2_ref, b2_ref, lw_ref, lb_ref, lin1_ref,
                 a_ref, h_ref):
    agg = part_ref[0] + part_ref[1]
    h2 = _ssp(jnp.dot(agg, w2_ref[...], preferred_element_type=jnp.float32)
              + b2_ref[...])
    a = jnp.dot(h2, lw_ref[...], preferred_element_type=jnp.float32) + lb_ref[...]
    a_ref[...] = a
    h_ref[...] = jnp.dot(a, lin1_ref[...], preferred_element_type=jnp.float32)


def _post_a(part, p, lin1_next, n, block=1000):
    grid = n // block
    vspec = pl.BlockSpec((1, D), lambda i: (0, 0))
    full = pl.BlockSpec((D, D), lambda i: (0, 0))
    return pl.pallas_call(
        _post_a_body,
        grid=(grid,),
        in_specs=[pl.BlockSpec((2, block, D), lambda i: (0, i, 0)),
                  full, vspec, full, vspec, full],
        out_specs=[pl.BlockSpec((block, D), lambda i: (i, 0)),
                   pl.BlockSpec((block, D), lambda i: (i, 0))],
        out_shape=[jax.ShapeDtypeStruct((n, D), jnp.float32),
                   jax.ShapeDtypeStruct((n, D), jnp.float32)],
    )(part, p['lin2_w'], p['lin2_b'].reshape(1, D),
      p['lin_w'], p['lin_b'].reshape(1, D), lin1_next)


def _final_body(n_valid, block, part_ref, w2_ref, b2_ref, lw_ref, lb_ref,
                g_ref, bn_ref, base_ref, delta_ref, o_ref):
    i = pl.program_id(0)
    row = lax.broadcasted_iota(jnp.int32, (block, 1), 0) + i * block
    agg = jnp.where(row < n_valid, part_ref[0] + part_ref[1], 0.0)
    h2 = _ssp(jnp.dot(agg, w2_ref[...], preferred_element_type=jnp.float32)
              + b2_ref[...])
    msg = jnp.dot(h2, lw_ref[...], preferred_element_type=jnp.float32) + lb_ref[...]
    msg = _ln(msg, g_ref[...], bn_ref[...])
    o_ref[...] = base_ref[...] + msg + delta_ref[...]


def _final(part, p, pln, base, delta, n, block):
    grid = n // block
    n_valid = part.shape[1]
    maxblk = (n_valid - 1) // block
    vspec = pl.BlockSpec((1, D), lambda i: (0, 0))
    full = pl.BlockSpec((D, D), lambda i: (0, 0))
    rows = pl.BlockSpec((block, D), lambda i: (i, 0))
    return pl.pallas_call(
        functools.partial(_final_body, n_valid, block),
        grid=(grid,),
        in_specs=[pl.BlockSpec((2, block, D),
                               lambda i: (0, jnp.minimum(i, maxblk), 0)),
                  full, vspec, full, vspec, vspec, vspec, rows, rows],
        out_specs=rows,
        out_shape=jax.ShapeDtypeStruct((n, D), jnp.float32),
    )(part, p['lin2_w'], p['lin2_b'].reshape(1, D),
      p['lin_w'], p['lin_b'].reshape(1, D),
      pln['g'].reshape(1, D), pln['b'].reshape(1, D), base, delta)


def _sc_agg(h, w, src, dst, z128, n_out, e_pad):
    chunk = 128 if n_out <= 2048 else 64
    ntot = e_pad // (16 * chunk)
    n0 = int(round(ntot * 0.62 / 2)) * 2
    n1 = ntot - n0
    assert n1 % 2 == 0 and n1 > 2
    n_pad = -(-n_out // 128) * 128
    rows_per = n_pad // 16

    mesh = plsc.VectorSubcoreMesh(core_axis_name="c", subcore_axis_name="s")

    @functools.partial(
        pl.kernel,
        out_type=jax.ShapeDtypeStruct((2, n_pad, D), jnp.float32),
        mesh=mesh,
        scratch_types=[
            pltpu.VMEM_SHARED((n_pad, D), jnp.float32),
            pltpu.VMEM((chunk,), jnp.int32),
            pltpu.VMEM((chunk,), jnp.int32),
            pltpu.VMEM((chunk,), jnp.int32),
            pltpu.VMEM((chunk,), jnp.int32),
            pltpu.VMEM((chunk, D), jnp.float32),
            pltpu.VMEM((chunk, D), jnp.float32),
            pltpu.VMEM((chunk, D), jnp.float32),
            pltpu.VMEM((chunk, D), jnp.float32),
            pltpu.SemaphoreType.DMA,
            pltpu.SemaphoreType.DMA,
            pltpu.SemaphoreType.DMA,
            pltpu.SemaphoreType.DMA,
            pltpu.SemaphoreType.DMA,
            pltpu.SemaphoreType.DMA,
        ],
    )
    def body(h_hbm, w_hbm, src_hbm, dst_hbm, z_hbm, out_hbm,
             acc_sh, src0, src1, dst0, dst1, rows0, rows1, wv0, wv1,
             semi0, semi1, semg0, semg1, semw0, semw1):
        c = lax.axis_index("c")
        s = lax.axis_index("s")
        myn = jnp.where(c == 0, n0, n1)
        srcb = (src0, src1)
        dstb = (dst0, dst1)
        rowsb = (rows0, rows1)
        wvb = (wv0, wv1)
        semi = (semi0, semi1)
        semg = (semg0, semg1)
        semw = (semw0, semw1)

        r0 = s * rows_per
        off = 0
        while off < rows_per:
            nrow = min(128, rows_per - off)
            pltpu.sync_copy(z_hbm.at[pl.ds(0, nrow)],
                            acc_sh.at[pl.ds(r0 + off, nrow)])
            off += nrow
        plsc.subcore_barrier()

        ebase = (s * ntot + c * n0) * chunk

        def issue_idx(k, b):
            pltpu.async_copy(src_hbm.at[pl.ds(ebase + k * chunk, chunk)],
                             srcb[b], semi[b])
            pltpu.async_copy(dst_hbm.at[pl.ds(ebase + k * chunk, chunk)],
                             dstb[b], semi[b])

        def wait_idx(k, b):
            pltpu.make_async_copy(src_hbm.at[pl.ds(ebase + k * chunk, chunk)],
                                  srcb[b], semi[b]).wait()
            pltpu.make_async_copy(dst_hbm.at[pl.ds(ebase + k * chunk, chunk)],
                                  dstb[b], semi[b]).wait()

        def issue_gw(k, b):
            pltpu.async_copy(h_hbm.at[srcb[b]], rowsb[b], semg[b])
            pltpu.async_copy(w_hbm.at[pl.ds(ebase + k * chunk, chunk)],
                             wvb[b], semw[b])

        def consume(k, b):
            pltpu.make_async_copy(h_hbm.at[srcb[b]], rowsb[b], semg[b]).wait()
            pltpu.make_async_copy(w_hbm.at[pl.ds(ebase + k * chunk, chunk)],
                                  wvb[b], semw[b]).wait()

            @plsc.parallel_loop(0, chunk, unroll=4)
            def _(e):
                for j in range(D // 16):
                    sl = pl.ds(j * 16, 16)
                    rowsb[b][e, sl] = rowsb[b][e, sl] * wvb[b][e, sl]

            pltpu.sync_copy(rowsb[b], acc_sh.at[dstb[b]], add=True)

        pltpu.sync_copy(src_hbm.at[pl.ds(ebase, chunk)], src0)
        pltpu.sync_copy(dst_hbm.at[pl.ds(ebase, chunk)], dst0)
        issue_gw(0, 0)
        issue_idx(1, 1)

        def step(k, b):
            bn = 1 - b

            @pl.when(k + 1 < myn)
            def _():
                wait_idx(k + 1, bn)
                issue_gw(k + 1, bn)

            consume(k, b)

            @pl.when(k + 2 < myn)
            def _():
                issue_idx(k + 2, b)

        def pair(j, carry):
            step(2 * j, 0)
            step(2 * j + 1, 1)
            return carry

        lax.fori_loop(0, myn // 2, pair, 0)

        plsc.subcore_barrier()
        pltpu.sync_copy(acc_sh.at[pl.ds(s * rows_per, rows_per)],
                        out_hbm.at[c, pl.ds(s * rows_per, rows_per)])

    return body(h, w, src, dst, z128)


def _sc_agg2(ha, wa, sa, da, hb, wb, sb, db, z128, n_out, e_pad):
    chunk = 128
    ntot = e_pad // (16 * chunk)
    n0 = int(round(ntot * 0.70 / 2)) * 2
    n1 = ntot - n0
    assert n1 % 2 == 0 and n1 >= 2
    n_pad = -(-n_out // 128) * 128
    rows_per = n_pad // 16

    mesh = plsc.VectorSubcoreMesh(core_axis_name="c", subcore_axis_name="s")

    @functools.partial(
        pl.kernel,
        out_type=[jax.ShapeDtypeStruct((2, n_pad, D), jnp.float32),
                  jax.ShapeDtypeStruct((2, n_pad, D), jnp.float32)],
        mesh=mesh,
        scratch_types=[
            pltpu.VMEM_SHARED((n_pad, D), jnp.float32),
            pltpu.VMEM_SHARED((n_pad, D), jnp.float32),
            pltpu.VMEM((chunk,), jnp.int32),
            pltpu.VMEM((chunk,), jnp.int32),
            pltpu.VMEM((chunk,), jnp.int32),
            pltpu.VMEM((chunk,), jnp.int32),
            pltpu.VMEM((chunk, D), jnp.float32),
            pltpu.VMEM((chunk, D), jnp.float32),
            pltpu.VMEM((chunk, D), jnp.float32),
            pltpu.VMEM((chunk, D), jnp.float32),
            pltpu.SemaphoreType.DMA,
            pltpu.SemaphoreType.DMA,
            pltpu.SemaphoreType.DMA,
            pltpu.SemaphoreType.DMA,
            pltpu.SemaphoreType.DMA,
            pltpu.SemaphoreType.DMA,
        ],
    )
    def body(ha_h, wa_h, sa_h, da_h, hb_h, wb_h, sb_h, db_h, z_hbm,
             oa_h, ob_h, acca, accb, src0, src1, dst0, dst1,
             rows0, rows1, wv0, wv1, semi0, semi1, semg0, semg1,
             semw0, semw1):
        c = lax.axis_index("c")
        s = lax.axis_index("s")
        myn = jnp.where(c == 0, n0, n1)
        srcb = (src0, src1)
        dstb = (dst0, dst1)
        rowsb = (rows0, rows1)
        wvb = (wv0, wv1)
        semi = (semi0, semi1)
        semg = (semg0, semg1)
        semw = (semw0, semw1)

        r0 = s * rows_per
        pltpu.sync_copy(z_hbm.at[pl.ds(0, rows_per)],
                        acca.at[pl.ds(r0, rows_per)])
        pltpu.sync_copy(z_hbm.at[pl.ds(0, rows_per)],
                        accb.at[pl.ds(r0, rows_per)])
        plsc.subcore_barrier()

        ebase = (s * ntot + c * n0) * chunk

        def run_set(h_hbm, w_hbm, src_hbm, dst_hbm, acc_sh):
            def issue_idx(k, b):
                pltpu.async_copy(src_hbm.at[pl.ds(ebase + k * chunk, chunk)],
                                 srcb[b], semi[b])
                pltpu.async_copy(dst_hbm.at[pl.ds(ebase + k * chunk, chunk)],
                                 dstb[b], semi[b])

            def wait_idx(k, b):
                pltpu.make_async_copy(
                    src_hbm.at[pl.ds(ebase + k * chunk, chunk)],
                    srcb[b], semi[b]).wait()
                pltpu.make_async_copy(
                    dst_hbm.at[pl.ds(ebase + k * chunk, chunk)],
                    dstb[b], semi[b]).wait()

            def issue_gw(k, b):
                pltpu.async_copy(h_hbm.at[srcb[b]], rowsb[b], semg[b])
                pltpu.async_copy(w_hbm.at[pl.ds(ebase + k * chunk, chunk)],
                                 wvb[b], semw[b])

            def consume(k, b):
                pltpu.make_async_copy(h_hbm.at[srcb[b]], rowsb[b],
                                      semg[b]).wait()
                pltpu.make_async_copy(
                    w_hbm.at[pl.ds(ebase + k * chunk, chunk)],
                    wvb[b], semw[b]).wait()

                @plsc.parallel_loop(0, chunk, unroll=4)
                def _(e):
                    for j in range(D // 16):
                        sl = pl.ds(j * 16, 16)
                        rowsb[b][e, sl] = rowsb[b][e, sl] * wvb[b][e, sl]

                pltpu.sync_copy(rowsb[b], acc_sh.at[dstb[b]], add=True)

            pltpu.sync_copy(src_hbm.at[pl.ds(ebase, chunk)], src0)
            pltpu.sync_copy(dst_hbm.at[pl.ds(ebase, chunk)], dst0)
            issue_gw(0, 0)
            issue_idx(1, 1)

            def step(k, b):
                bn = 1 - b

                @pl.when(k + 1 < myn)
                def _():
                    wait_idx(k + 1, bn)
                    issue_gw(k + 1, bn)

                consume(k, b)

                @pl.when(k + 2 < myn)
                def _():
                    issue_idx(k + 2, b)

            def pair(j, carry):
                step(2 * j, 0)
                step(2 * j + 1, 1)
                return carry

            lax.fori_loop(0, myn // 2, pair, 0)

        run_set(ha_h, wa_h, sa_h, da_h, acca)
        run_set(hb_h, wb_h, sb_h, db_h, accb)

        plsc.subcore_barrier()
        pltpu.sync_copy(acca.at[pl.ds(r0, rows_per)],
                        oa_h.at[c, pl.ds(r0, rows_per)])
        pltpu.sync_copy(accb.at[pl.ds(r0, rows_per)],
                        ob_h.at[c, pl.ds(r0, rows_per)])

    return body(ha, wa, sa, da, hb, wb, sb, db, z128)


def _pad_idx(idx, e_pad):
    e = idx.shape[1]
    src = jnp.pad(idx[0], (0, e_pad - e))
    dst = jnp.pad(idx[1], (0, e_pad - e))
    return src, dst


def kernel(a_x, m_x, a2a_edge_index, a2m_edge_index, m2a_edge_index,
           a2a_edge_weights, a2m_edge_weights, m2a_edge_weights,
           a2a_edge_attr, a2m_edge_attr, m2a_edge_attr, params):
    n_a = a_x.shape[0]
    n_m = m_x.shape[0]
    z128 = jnp.zeros((128, D), jnp.float32)

    aa_pad = -(-a2a_edge_attr.shape[0] // (NW * CHUNK)) * (NW * CHUNK)
    am_pad = -(-a2m_edge_attr.shape[0] // (NW * CHUNK)) * (NW * CHUNK)
    ma_pad = -(-m2a_edge_attr.shape[0] // (NW * CHUNK)) * (NW * CHUNK)
    aa_src, aa_dst = _pad_idx(a2a_edge_index, aa_pad)
    am_src, am_dst = _pad_idx(a2m_edge_index, am_pad)
    ma_src, ma_dst = _pad_idx(m2a_edge_index, ma_pad)

    w_aa = _edge_w(a2a_edge_attr, a2a_edge_weights, params['short'], aa_pad)
    w_am = _edge_w(a2m_edge_attr, a2m_edge_weights, params['a2m'], am_pad)
    w_ma = _edge_w(m2a_edge_attr, m2a_edge_weights, params['m2a'], ma_pad)

    h_short = _pre_a(a_x, params['ln_short']['g'], params['ln_short']['b'],
                     params['short']['lin1_w'])

    m, h_m2a = _mha(m_x, params['ln_long'], params['mha'],
                    params['m2a']['lin1_w'])

    part_aa = _sc_agg(h_short, w_aa, aa_src, aa_dst, z128, n_a, aa_pad)

    a, h_a2m = _post_a(part_aa, params['short'], params['a2m']['lin1_w'], n_a)

    part_am, part_ma = _sc_agg2(h_a2m, w_am, am_src, am_dst,
                                h_m2a, w_ma, ma_src, ma_dst,
                                z128, n_m, am_pad)

    a_out = _final(part_ma, params['m2a'], params['ln_m2a'], a, a_x, n_a, 1000)
    m_out = _final(part_am, params['a2m'], params['ln_a2m'], m, m_x, n_m, 512)
    return (a_out, m_out)


# speedup vs baseline: 3.6673x; 1.2076x over previous
import functools
import math

import jax
import jax.numpy as jnp
from jax import lax
from jax.experimental import pallas as pl
from jax.experimental.pallas import tpu as pltpu
from jax.experimental.pallas import tpu_sc as plsc

D = 128
R = 50
NHEAD = 8
DH = D // NHEAD
LOG2 = math.log(2.0)
EPS = 1e-5

NW = 32
CHUNK = 128
BE = 4096


def _ssp(x):
    return jnp.maximum(x, 0.0) + jnp.log(1.0 + jnp.exp(-jnp.abs(x))) - LOG2


def _ln(x, g, b):
    mu = jnp.mean(x, axis=-1, keepdims=True)
    var = jnp.mean((x - mu) ** 2, axis=-1, keepdims=True)
    return (x - mu) * lax.rsqrt(var + EPS) * g + b


def _pre_a_body(x_ref, g_ref, b_ref, w_ref, o_ref):
    x = _ln(x_ref[...], g_ref[...], b_ref[...])
    o_ref[...] = jnp.dot(x, w_ref[...], preferred_element_type=jnp.float32)


def _pre_a(x, g, b, w, block=1000):
    n = x.shape[0]
    grid = n // block
    return pl.pallas_call(
        _pre_a_body,
        grid=(grid,),
        in_specs=[
            pl.BlockSpec((block, D), lambda i: (i, 0)),
            pl.BlockSpec((1, D), lambda i: (0, 0)),
            pl.BlockSpec((1, D), lambda i: (0, 0)),
            pl.BlockSpec((D, D), lambda i: (0, 0)),
        ],
        out_specs=pl.BlockSpec((block, D), lambda i: (i, 0)),
        out_shape=jax.ShapeDtypeStruct((n, D), jnp.float32),
    )(x, g.reshape(1, D), b.reshape(1, D), w)


def _edge_w_body(n_valid, attr_ref, ew_ref, w1_ref, b1_ref, w2_ref,
                 b2_ref, o_ref):
    i = pl.program_id(0)
    t = _ssp(lax.dot_general(attr_ref[...], w1_ref[...],
                             (((0,), (0,)), ((), ())),
                             preferred_element_type=jnp.float32) + b1_ref[...])
    w = jnp.dot(t, w2_ref[...], preferred_element_type=jnp.float32) + b2_ref[...]
    cblk = 0.5 * (jnp.cos(ew_ref[...] * jnp.pi) + 1.0)
    w = (w.reshape(BE // 128, 128, D) * cblk[:, :, None]).reshape(BE, D)
    row = lax.broadcasted_iota(jnp.int32, (BE, 1), 0) + i * BE
    o_ref[...] = jnp.where(row < n_valid, w, 0.0)


def _edge_w(attr, ew, p, e_pad):
    n_valid = attr.shape[0]
    grid = e_pad // BE
    ew2 = ew.reshape(-1, 128)
    attr_t = attr.T
    return pl.pallas_call(
        functools.partial(_edge_w_body, n_valid),
        grid=(grid,),
        in_specs=[
            pl.BlockSpec((R, BE), lambda i: (0, i)),
            pl.BlockSpec((BE // 128, 128), lambda i: (i, 0)),
            pl.BlockSpec((R, D), lambda i: (0, 0)),
            pl.BlockSpec((1, D), lambda i: (0, 0)),
            pl.BlockSpec((D, D), lambda i: (0, 0)),
            pl.BlockSpec((1, D), lambda i: (0, 0)),
        ],
        out_specs=pl.BlockSpec((BE, D), lambda i: (i, 0)),
        out_shape=jax.ShapeDtypeStruct((e_pad, D), jnp.float32),
    )(attr_t, ew2, p['mlp_w1'], p['mlp_b1'].reshape(1, D),
      p['mlp_w2'], p['mlp_b2'].reshape(1, D))


def _mha_body(x_ref, g_ref, b_ref, wq_ref, bq_ref, wk_ref, bk_ref, wv_ref,
              bv_ref, wo_ref, bo_ref, lin1_ref, m_ref, h_ref):
    x = _ln(x_ref[...], g_ref[...], b_ref[...])
    q = jnp.dot(x, wq_ref[...], preferred_element_type=jnp.float32) + bq_ref[...]
    k = jnp.dot(x, wk_ref[...], preferred_element_type=jnp.float32) + bk_ref[...]
    v = jnp.dot(x, wv_ref[...], preferred_element_type=jnp.float32) + bv_ref[...]
    outs = []
    for h in range(NHEAD):
        sl = slice(h * DH, (h + 1) * DH)
        qh, kh, vh = q[:, sl], k[:, sl], v[:, sl]
        att = lax.dot_general(qh, kh, (((1,), (1,)), ((), ())),
                              preferred_element_type=jnp.float32)
        att = jax.nn.softmax(att * (1.0 / math.sqrt(DH)), axis=-1)
        outs.append(jnp.dot(att, vh, preferred_element_type=jnp.float32))
    o = jnp.concatenate(outs, axis=-1)
    m = jnp.dot(o, wo_ref[...], preferred_element_type=jnp.float32) + bo_ref[...]
    m_ref[...] = m
    h_ref[...] = jnp.dot(m, lin1_ref[...], preferred_element_type=jnp.float32)


def _mha(m_x, pm, pmha, lin1_w, seq=512):
    n = m_x.shape[0]
    grid = n // seq
    vec = lambda a: a.reshape(1, D)
    full = pl.BlockSpec((D, D), lambda i: (0, 0))
    vspec = pl.BlockSpec((1, D), lambda i: (0, 0))
    return pl.pallas_call(
        _mha_body,
        grid=(grid,),
        in_specs=[pl.BlockSpec((seq, D), lambda i: (i, 0)),
                  vspec, vspec,
                  full, vspec, full, vspec, full, vspec, full, vspec, full],
        out_specs=[pl.BlockSpec((seq, D), lambda i: (i, 0)),
                   pl.BlockSpec((seq, D), lambda i: (i, 0))],
        out_shape=[jax.ShapeDtypeStruct((n, D), jnp.float32),
                   jax.ShapeDtypeStruct((n, D), jnp.float32)],
    )(m_x, vec(pm['g']), vec(pm['b']),
      pmha['wq'], vec(pmha['bq']), pmha['wk'], vec(pmha['bk']),
      pmha['wv'], vec(pmha['bv']), pmha['wo'], vec(pmha['bo']), lin1_w)


def _post_a_body(part_ref, w2_ref, b2_ref, lw_ref, lb_ref, lin1_ref,
                 a_ref, h_ref):
    agg = part_ref[0] + part_ref[1]
    h2 = _ssp(jnp.dot(agg, w2_ref[...], preferred_element_type=jnp.float32)
              + b2_ref[...])
    a = jnp.dot(h2, lw_ref[...], preferred_element_type=jnp.float32) + lb_ref[...]
    a_ref[...] = a
    h_ref[...] = jnp.dot(a, lin1_ref[...], preferred_element_type=jnp.float32)


def _post_a(part, p, lin1_next, n, block=1000):
    grid = n // block
    vspec = pl.BlockSpec((1, D), lambda i: (0, 0))
    full = pl.BlockSpec((D, D), lambda i: (0, 0))
    return pl.pallas_call(
        _post_a_body,
        grid=(grid,),
        in_specs=[pl.BlockSpec((2, block, D), lambda i: (0, i, 0)),
                  full, vspec, full, vspec, full],
        out_specs=[pl.BlockSpec((block, D), lambda i: (i, 0)),
                   pl.BlockSpec((block, D), lambda i: (i, 0))],
        out_shape=[jax.ShapeDtypeStruct((n, D), jnp.float32),
                   jax.ShapeDtypeStruct((n, D), jnp.float32)],
    )(part, p['lin2_w'], p['lin2_b'].reshape(1, D),
      p['lin_w'], p['lin_b'].reshape(1, D), lin1_next)


def _final_body(n_valid, block, part_ref, w2_ref, b2_ref, lw_ref, lb_ref,
                g_ref, bn_ref, base_ref, delta_ref, o_ref):
    i = pl.program_id(0)
    row = lax.broadcasted_iota(jnp.int32, (block, 1), 0) + i * block
    agg = jnp.where(row < n_valid, part_ref[0] + part_ref[1], 0.0)
    h2 = _ssp(jnp.dot(agg, w2_ref[...], preferred_element_type=jnp.float32)
              + b2_ref[...])
    msg = jnp.dot(h2, lw_ref[...], preferred_element_type=jnp.float32) + lb_ref[...]
    msg = _ln(msg, g_ref[...], bn_ref[...])
    o_ref[...] = base_ref[...] + msg + delta_ref[...]


def _final(part, p, pln, base, delta, n, block):
    grid = n // block
    n_valid = part.shape[1]
    maxblk = (n_valid - 1) // block
    vspec = pl.BlockSpec((1, D), lambda i: (0, 0))
    full = pl.BlockSpec((D, D), lambda i: (0, 0))
    rows = pl.BlockSpec((block, D), lambda i: (i, 0))
    return pl.pallas_call(
        functools.partial(_final_body, n_valid, block),
        grid=(grid,),
        in_specs=[pl.BlockSpec((2, block, D),
                               lambda i: (0, jnp.minimum(i, maxblk), 0)),
                  full, vspec, full, vspec, vspec, vspec, rows, rows],
        out_specs=rows,
        out_shape=jax.ShapeDtypeStruct((n, D), jnp.float32),
    )(part, p['lin2_w'], p['lin2_b'].reshape(1, D),
      p['lin_w'], p['lin_b'].reshape(1, D),
      pln['g'].reshape(1, D), pln['b'].reshape(1, D), base, delta)


def _sc_agg(h, w, src, dst, z128, n_out, e_pad):
    chunk = 128 if n_out <= 2048 else 64
    ntot = e_pad // (16 * chunk)
    n0 = int(round(ntot * 0.62 / 2)) * 2
    n1 = ntot - n0
    assert n1 % 2 == 0 and n1 > 2
    n_pad = -(-n_out // 128) * 128
    rows_per = n_pad // 16

    mesh = plsc.VectorSubcoreMesh(core_axis_name="c", subcore_axis_name="s")

    @functools.partial(
        pl.kernel,
        out_type=jax.ShapeDtypeStruct((2, n_pad, D), jnp.float32),
        mesh=mesh,
        scratch_types=[
            pltpu.VMEM_SHARED((n_pad, D), jnp.float32),
            pltpu.VMEM((chunk,), jnp.int32),
            pltpu.VMEM((chunk,), jnp.int32),
            pltpu.VMEM((chunk,), jnp.int32),
            pltpu.VMEM((chunk,), jnp.int32),
            pltpu.VMEM((chunk, D), jnp.float32),
            pltpu.VMEM((chunk, D), jnp.float32),
            pltpu.VMEM((chunk, D), jnp.float32),
            pltpu.VMEM((chunk, D), jnp.float32),
            pltpu.SemaphoreType.DMA,
            pltpu.SemaphoreType.DMA,
            pltpu.SemaphoreType.DMA,
            pltpu.SemaphoreType.DMA,
            pltpu.SemaphoreType.DMA,
            pltpu.SemaphoreType.DMA,
        ],
    )
    def body(h_hbm, w_hbm, src_hbm, dst_hbm, z_hbm, out_hbm,
             acc_sh, src0, src1, dst0, dst1, rows0, rows1, wv0, wv1,
             semi0, semi1, semg0, semg1, semw0, semw1):
        c = lax.axis_index("c")
        s = lax.axis_index("s")
        myn = jnp.where(c == 0, n0, n1)
        srcb = (src0, src1)
        dstb = (dst0, dst1)
        rowsb = (rows0, rows1)
        wvb = (wv0, wv1)
        semi = (semi0, semi1)
        semg = (semg0, semg1)
        semw = (semw0, semw1)

        r0 = s * rows_per
        off = 0
        while off < rows_per:
            nrow = min(128, rows_per - off)
            pltpu.sync_copy(z_hbm.at[pl.ds(0, nrow)],
                            acc_sh.at[pl.ds(r0 + off, nrow)])
            off += nrow
        plsc.subcore_barrier()

        ebase = (s * ntot + c * n0) * chunk

        def issue_idx(k, b):
            pltpu.async_copy(src_hbm.at[pl.ds(ebase + k * chunk, chunk)],
                             srcb[b], semi[b])
            pltpu.async_copy(dst_hbm.at[pl.ds(ebase + k * chunk, chunk)],
                             dstb[b], semi[b])

        def wait_idx(k, b):
            pltpu.make_async_copy(src_hbm.at[pl.ds(ebase + k * chunk, chunk)],
                                  srcb[b], semi[b]).wait()
            pltpu.make_async_copy(dst_hbm.at[pl.ds(ebase + k * chunk, chunk)],
                                  dstb[b], semi[b]).wait()

        def issue_gw(k, b):
            pltpu.async_copy(h_hbm.at[srcb[b]], rowsb[b], semg[b])
            pltpu.async_copy(w_hbm.at[pl.ds(ebase + k * chunk, chunk)],
                             wvb[b], semw[b])

        def consume(k, b):
            pltpu.make_async_copy(h_hbm.at[srcb[b]], rowsb[b], semg[b]).wait()
            pltpu.make_async_copy(w_hbm.at[pl.ds(ebase + k * chunk, chunk)],
                                  wvb[b], semw[b]).wait()

            @plsc.parallel_loop(0, chunk, unroll=4)
            def _(e):
                for j in range(D // 16):
                    sl = pl.ds(j * 16, 16)
                    rowsb[b][e, sl] = rowsb[b][e, sl] * wvb[b][e, sl]

            pltpu.sync_copy(rowsb[b], acc_sh.at[dstb[b]], add=True)

        pltpu.sync_copy(src_hbm.at[pl.ds(ebase, chunk)], src0)
        pltpu.sync_copy(dst_hbm.at[pl.ds(ebase, chunk)], dst0)
        issue_gw(0, 0)
        issue_idx(1, 1)

        def step(k, b):
            bn = 1 - b

            @pl.when(k + 1 < myn)
            def _():
                wait_idx(k + 1, bn)
                issue_gw(k + 1, bn)

            consume(k, b)

            @pl.when(k + 2 < myn)
            def _():
                issue_idx(k + 2, b)

        def pair(j, carry):
            step(2 * j, 0)
            step(2 * j + 1, 1)
            return carry

        lax.fori_loop(0, myn // 2, pair, 0)

        plsc.subcore_barrier()
        pltpu.sync_copy(acc_sh.at[pl.ds(s * rows_per, rows_per)],
                        out_hbm.at[c, pl.ds(s * rows_per, rows_per)])

    return body(h, w, src, dst, z128)


def _sc_agg2(ha, wa, sa, da, hb, wb, sb, db, z128, n_out, e_pad):
    chunk = 128
    ntot = e_pad // (16 * chunk)
    n0 = int(round(ntot * 0.70 / 2)) * 2
    n1 = ntot - n0
    assert n1 % 2 == 0 and n1 >= 2
    n_pad = -(-n_out // 128) * 128
    rows_per = n_pad // 16

    mesh = plsc.VectorSubcoreMesh(core_axis_name="c", subcore_axis_name="s")

    @functools.partial(
        pl.kernel,
        out_type=[jax.ShapeDtypeStruct((2, n_pad, D), jnp.float32),
                  jax.ShapeDtypeStruct((2, n_pad, D), jnp.float32)],
        mesh=mesh,
        scratch_types=[
            pltpu.VMEM_SHARED((n_pad, D), jnp.float32),
            pltpu.VMEM_SHARED((n_pad, D), jnp.float32),
            pltpu.VMEM((chunk,), jnp.int32),
            pltpu.VMEM((chunk,), jnp.int32),
            pltpu.VMEM((chunk,), jnp.int32),
            pltpu.VMEM((chunk,), jnp.int32),
            pltpu.VMEM((chunk, D), jnp.float32),
            pltpu.VMEM((chunk, D), jnp.float32),
            pltpu.VMEM((chunk, D), jnp.float32),
            pltpu.VMEM((chunk, D), jnp.float32),
            pltpu.SemaphoreType.DMA,
            pltpu.SemaphoreType.DMA,
            pltpu.SemaphoreType.DMA,
            pltpu.SemaphoreType.DMA,
            pltpu.SemaphoreType.DMA,
            pltpu.SemaphoreType.DMA,
        ],
    )
    def body(ha_h, wa_h, sa_h, da_h, hb_h, wb_h, sb_h, db_h, z_hbm,
             oa_h, ob_h, acca, accb, src0, src1, dst0, dst1,
             rows0, rows1, wv0, wv1, semi0, semi1, semg0, semg1,
             semw0, semw1):
        c = lax.axis_index("c")
        s = lax.axis_index("s")
        myn = jnp.where(c == 0, n0, n1)
        srcb = (src0, src1)
        dstb = (dst0, dst1)
        rowsb = (rows0, rows1)
        wvb = (wv0, wv1)
        semi = (semi0, semi1)
        semg = (semg0, semg1)
        semw = (semw0, semw1)

        r0 = s * rows_per
        pltpu.sync_copy(z_hbm.at[pl.ds(0, rows_per)],
                        acca.at[pl.ds(r0, rows_per)])
        pltpu.sync_copy(z_hbm.at[pl.ds(0, rows_per)],
                        accb.at[pl.ds(r0, rows_per)])
        plsc.subcore_barrier()

        ebase = (s * ntot + c * n0) * chunk

        def run_set(h_hbm, w_hbm, src_hbm, dst_hbm, acc_sh):
            def issue_idx(k, b):
                pltpu.async_copy(src_hbm.at[pl.ds(ebase + k * chunk, chunk)],
                                 srcb[b], semi[b])
                pltpu.async_copy(dst_hbm.at[pl.ds(ebase + k * chunk, chunk)],
                                 dstb[b], semi[b])

            def wait_idx(k, b):
                pltpu.make_async_copy(
                    src_hbm.at[pl.ds(ebase + k * chunk, chunk)],
                    srcb[b], semi[b]).wait()
                pltpu.make_async_copy(
                    dst_hbm.at[pl.ds(ebase + k * chunk, chunk)],
                    dstb[b], semi[b]).wait()

            def issue_gw(k, b):
                pltpu.async_copy(h_hbm.at[srcb[b]], rowsb[b], semg[b])
                pltpu.async_copy(w_hbm.at[pl.ds(ebase + k * chunk, chunk)],
                                 wvb[b], semw[b])

            def consume(k, b):
                pltpu.make_async_copy(h_hbm.at[srcb[b]], rowsb[b],
                                      semg[b]).wait()
                pltpu.make_async_copy(
                    w_hbm.at[pl.ds(ebase + k * chunk, chunk)],
                    wvb[b], semw[b]).wait()

                @plsc.parallel_loop(0, chunk, unroll=4)
                def _(e):
                    for j in range(D // 16):
                        sl = pl.ds(j * 16, 16)
                        rowsb[b][e, sl] = rowsb[b][e, sl] * wvb[b][e, sl]

                pltpu.sync_copy(rowsb[b], acc_sh.at[dstb[b]], add=True)

            pltpu.sync_copy(src_hbm.at[pl.ds(ebase, chunk)], src0)
            pltpu.sync_copy(dst_hbm.at[pl.ds(ebase, chunk)], dst0)
            issue_gw(0, 0)
            issue_idx(1, 1)

            def step(k, b):
                bn = 1 - b

                @pl.when(k + 1 < myn)
                def _():
                    wait_idx(k + 1, bn)
                    issue_gw(k + 1, bn)

                consume(k, b)

                @pl.when(k + 2 < myn)
                def _():
                    issue_idx(k + 2, b)

            def pair(j, carry):
                step(2 * j, 0)
                step(2 * j + 1, 1)
                return carry

            lax.fori_loop(0, myn // 2, pair, 0)

        run_set(ha_h, wa_h, sa_h, da_h, acca)
        run_set(hb_h, wb_h, sb_h, db_h, accb)

        plsc.subcore_barrier()
        pltpu.sync_copy(acca.at[pl.ds(r0, rows_per)],
                        oa_h.at[c, pl.ds(r0, rows_per)])
        pltpu.sync_copy(accb.at[pl.ds(r0, rows_per)],
                        ob_h.at[c, pl.ds(r0, rows_per)])

    return body(ha, wa, sa, da, hb, wb, sb, db, z128)


def _pad_idx(idx, e_pad):
    e = idx.shape[1]
    src = jnp.pad(idx[0], (0, e_pad - e))
    dst = jnp.pad(idx[1], (0, e_pad - e))
    return src, dst


def kernel(a_x, m_x, a2a_edge_index, a2m_edge_index, m2a_edge_index,
           a2a_edge_weights, a2m_edge_weights, m2a_edge_weights,
           a2a_edge_attr, a2m_edge_attr, m2a_edge_attr, params):
    n_a = a_x.shape[0]
    n_m = m_x.shape[0]
    z128 = jnp.zeros((128, D), jnp.float32)

    aa_pad = -(-a2a_edge_attr.shape[0] // (NW * CHUNK)) * (NW * CHUNK)
    am_pad = -(-a2m_edge_attr.shape[0] // (NW * CHUNK)) * (NW * CHUNK)
    ma_pad = -(-m2a_edge_attr.shape[0] // (NW * CHUNK)) * (NW * CHUNK)
    aa_src, aa_dst = _pad_idx(a2a_edge_index, aa_pad)
    am_src, am_dst = _pad_idx(a2m_edge_index, am_pad)
    ma_src, ma_dst = _pad_idx(m2a_edge_index, ma_pad)

    w_aa = _edge_w(a2a_edge_attr, a2a_edge_weights, params['short'], aa_pad)
    w_am = _edge_w(a2m_edge_attr, a2m_edge_weights, params['a2m'], am_pad)
    w_ma = _edge_w(m2a_edge_attr, m2a_edge_weights, params['m2a'], ma_pad)

    h_short = _pre_a(a_x, params['ln_short']['g'], params['ln_short']['b'],
                     params['short']['lin1_w'])

    m, h_m2a = _mha(m_x, params['ln_long'], params['mha'],
                    params['m2a']['lin1_w'])

    part_aa = _sc_agg(h_short, w_aa, aa_src, aa_dst, z128, n_a, aa_pad)

    a, h_a2m = _post_a(part_aa, params['short'], params['a2m']['lin1_w'], n_a)

    part_am, part_ma = _sc_agg2(h_a2m, w_am, am_src, am_dst,
                                h_m2a, w_ma, ma_src, ma_dst,
                                z128, n_m, am_pad)

    a_out = _final(part_ma, params['m2a'], params['ln_m2a'], a, a_x, n_a, 1000)
    m_out = _final(part_am, params['a2m'], params['ln_a2m'], m, m_x, n_m, 512)
    return (a_out, m_out)
